# Initial kernel scaffold; baseline (speedup 1.0000x reference)
import jax
import jax.numpy as jnp
from jax.experimental import pallas as pl


def kernel(x_true, x_pred, masks, ind_of_ind):
    raise NotImplementedError("write your pallas kernel here")


# speedup vs baseline: 3.7997x; 3.7997x over previous
import jax
import jax.numpy as jnp
from jax import lax
from jax.experimental import pallas as pl
from jax.experimental.pallas import tpu as pltpu

_MAX_LEN = 199
_DIM = 128
_NUM_MASKS = 32
_BB = 8


def _cl_kernel(ind_ref, masks_ref, xt_ref, xp_ref, out_ref):
    ind = ind_ref[...]
    kiota = lax.broadcasted_iota(jnp.int32, (_NUM_MASKS, _DIM), 0)
    onehot_kd = (kiota == jnp.broadcast_to(ind, (_NUM_MASKS, _DIM))).astype(
        jnp.float32
    )
    table = lax.dot_general(
        onehot_kd,
        masks_ref[...],
        dimension_numbers=(((0,), (0,)), ((), ())),
        preferred_element_type=jnp.float32,
    )

    xt = xt_ref[...]
    m = jnp.max(xt, axis=-1, keepdims=True)
    liota = lax.broadcasted_iota(jnp.int32, xt.shape, 2)
    idx = jnp.min(jnp.where(xt == m, liota, _DIM), axis=-1, keepdims=True)
    onehot = (liota == idx).astype(jnp.float32)

    rows = _BB * _MAX_LEN
    mask_rows = jnp.dot(
        onehot.reshape(rows, _DIM), table, preferred_element_type=jnp.float32
    )
    p = jnp.exp(xp_ref[...]).reshape(rows, _DIM) * mask_rows
    p = p / jnp.sum(p, axis=-1, keepdims=True)
    out_ref[...] = p.reshape(_BB, _MAX_LEN, _DIM)


@jax.jit
def kernel(x_true, x_pred, masks, ind_of_ind):
    batch = x_true.shape[0]
    grid = (batch // _BB,)
    return pl.pallas_call(
        _cl_kernel,
        grid=grid,
        in_specs=[
            pl.BlockSpec((1, _DIM), lambda b: (0, 0)),
            pl.BlockSpec((_NUM_MASKS, _DIM), lambda b: (0, 0)),
            pl.BlockSpec((_BB, _MAX_LEN, _DIM), lambda b: (b, 0, 0)),
            pl.BlockSpec((_BB, _MAX_LEN, _DIM), lambda b: (b, 0, 0)),
        ],
        out_specs=pl.BlockSpec((_BB, _MAX_LEN, _DIM), lambda b: (b, 0, 0)),
        out_shape=jax.ShapeDtypeStruct(x_true.shape, jnp.float32),
        compiler_params=pltpu.CompilerParams(
            dimension_semantics=("parallel",),
        ),
    )(ind_of_ind.reshape(1, _DIM).astype(jnp.int32), masks, x_true, x_pred)


# speedup vs baseline: 4.2393x; 1.1157x over previous
import jax
import jax.numpy as jnp
from jax import lax
from jax.experimental import pallas as pl
from jax.experimental.pallas import tpu as pltpu

_MAX_LEN = 199
_DIM = 128
_NUM_MASKS = 32
_RB = 1592


def _cl_kernel(ind_ref, masks_ref, xt_ref, xp_ref, out_ref):
    ind = ind_ref[...]
    kiota = lax.broadcasted_iota(jnp.int32, (_NUM_MASKS, _DIM), 0)
    onehot_kd = (kiota == jnp.broadcast_to(ind, (_NUM_MASKS, _DIM))).astype(
        jnp.float32
    )
    table = lax.dot_general(
        onehot_kd,
        masks_ref[...],
        dimension_numbers=(((0,), (0,)), ((), ())),
        preferred_element_type=jnp.float32,
    )

    xt = xt_ref[...]
    m = jnp.max(xt, axis=-1, keepdims=True)
    eqf = (xt == m).astype(jnp.float32)
    r = lax.broadcasted_iota(jnp.int32, (_DIM, _DIM), 0)
    c = lax.broadcasted_iota(jnp.int32, (_DIM, _DIM), 1)
    tri = (r < c).astype(jnp.float32)
    pf = jnp.dot(eqf, tri, preferred_element_type=jnp.float32)
    onehot = jnp.where(pf == 0.0, eqf, 0.0)

    mask_rows = jnp.dot(
        onehot, table, preferred_element_type=jnp.float32
    )
    p = jnp.exp(xp_ref[...]) * mask_rows
    denom = jnp.dot(
        p, jnp.ones((_DIM, _DIM), jnp.float32), preferred_element_type=jnp.float32
    )
    out_ref[...] = p / denom


@jax.jit
def kernel(x_true, x_pred, masks, ind_of_ind):
    batch, seq, dim = x_true.shape
    rows = batch * seq
    grid = (rows // _RB,)
    out = pl.pallas_call(
        _cl_kernel,
        grid=grid,
        in_specs=[
            pl.BlockSpec((1, _DIM), lambda b: (0, 0)),
            pl.BlockSpec((_NUM_MASKS, _DIM), lambda b: (0, 0)),
            pl.BlockSpec((_RB, _DIM), lambda b: (b, 0)),
            pl.BlockSpec((_RB, _DIM), lambda b: (b, 0)),
        ],
        out_specs=pl.BlockSpec((_RB, _DIM), lambda b: (b, 0)),
        out_shape=jax.ShapeDtypeStruct((rows, dim), jnp.float32),
        compiler_params=pltpu.CompilerParams(
            dimension_semantics=("parallel",),
        ),
    )(
        ind_of_ind.reshape(1, _DIM).astype(jnp.int32),
        masks,
        x_true.reshape(rows, dim),
        x_pred.reshape(rows, dim),
    )
    return out.reshape(batch, seq, dim)


# speedup vs baseline: 5.9544x; 1.4046x over previous
import jax
import jax.numpy as jnp
from jax import lax
from jax.experimental import pallas as pl
from jax.experimental.pallas import tpu as pltpu

_MAX_LEN = 199
_DIM = 128
_NUM_MASKS = 32
_BB = 128
_LB = 8


def _cl_kernel(ind_ref, masks_ref, xt_ref, xp_ref, out_ref):
    ind = ind_ref[...]
    kiota = lax.broadcasted_iota(jnp.int32, (_NUM_MASKS, _DIM), 0)
    onehot_kd = (kiota == jnp.broadcast_to(ind, (_NUM_MASKS, _DIM))).astype(
        jnp.float32
    )
    table = lax.dot_general(
        onehot_kd,
        masks_ref[...],
        dimension_numbers=(((0,), (0,)), ((), ())),
        preferred_element_type=jnp.float32,
    )

    rows = _BB * _LB
    xt = xt_ref[...].reshape(rows, _DIM)
    m = jnp.max(xt, axis=-1, keepdims=True)
    eqf = (xt == m).astype(jnp.float32)
    r = lax.broadcasted_iota(jnp.int32, (_DIM, _DIM), 0)
    c = lax.broadcasted_iota(jnp.int32, (_DIM, _DIM), 1)
    tri = (r < c).astype(jnp.float32)
    pf = jnp.dot(eqf, tri, preferred_element_type=jnp.float32)
    onehot = jnp.where(pf == 0.0, eqf, 0.0)

    mask_rows = jnp.dot(
        onehot, table, preferred_element_type=jnp.float32
    )
    p = jnp.exp(xp_ref[...]).reshape(rows, _DIM) * mask_rows
    denom = jnp.dot(
        p, jnp.ones((_DIM, _DIM), jnp.float32), preferred_element_type=jnp.float32
    )
    out_ref[...] = (p / denom).reshape(_BB, _LB, _DIM)


@jax.jit
def kernel(x_true, x_pred, masks, ind_of_ind):
    batch, seq, dim = x_true.shape
    grid = (batch // _BB, pl.cdiv(seq, _LB))
    return pl.pallas_call(
        _cl_kernel,
        grid=grid,
        in_specs=[
            pl.BlockSpec((1, _DIM), lambda b, l: (0, 0)),
            pl.BlockSpec((_NUM_MASKS, _DIM), lambda b, l: (0, 0)),
            pl.BlockSpec((_BB, _LB, _DIM), lambda b, l: (b, l, 0)),
            pl.BlockSpec((_BB, _LB, _DIM), lambda b, l: (b, l, 0)),
        ],
        out_specs=pl.BlockSpec((_BB, _LB, _DIM), lambda b, l: (b, l, 0)),
        out_shape=jax.ShapeDtypeStruct(x_true.shape, jnp.float32),
        compiler_params=pltpu.CompilerParams(
            dimension_semantics=("parallel", "parallel"),
        ),
    )(
        ind_of_ind.reshape(1, _DIM).astype(jnp.int32),
        masks,
        x_true,
        x_pred,
    )


# speedup vs baseline: 7.7227x; 1.2970x over previous
import jax
import jax.numpy as jnp
from jax import lax
from jax.experimental import pallas as pl
from jax.experimental.pallas import tpu as pltpu

_MAX_LEN = 199
_DIM = 128
_NUM_MASKS = 32
_BB = 1024
_LB = 8


def _cl_kernel(ind_ref, masks_ref, xt_ref, xp_ref, out_ref):
    ind = ind_ref[...]
    kiota = lax.broadcasted_iota(jnp.int32, (_NUM_MASKS, _DIM), 0)
    onehot_kd = (kiota == jnp.broadcast_to(ind, (_NUM_MASKS, _DIM))).astype(
        jnp.float32
    )
    table = lax.dot_general(
        onehot_kd,
        masks_ref[...],
        dimension_numbers=(((0,), (0,)), ((), ())),
        preferred_element_type=jnp.float32,
    )

    rows = _BB * _LB
    xt = xt_ref[...].reshape(rows, _DIM)
    m = jnp.max(xt, axis=-1, keepdims=True)
    eqf = (xt == m).astype(jnp.float32)
    r = lax.broadcasted_iota(jnp.int32, (_DIM, _DIM), 0)
    c = lax.broadcasted_iota(jnp.int32, (_DIM, _DIM), 1)
    tri = (r < c).astype(jnp.float32)
    pf = jnp.dot(eqf, tri, preferred_element_type=jnp.float32)
    onehot = jnp.where(pf == 0.0, eqf, 0.0)

    mask_rows = jnp.dot(
        onehot, table, preferred_element_type=jnp.float32
    )
    p = jnp.exp(xp_ref[...]).reshape(rows, _DIM) * mask_rows
    denom = jnp.dot(
        p, jnp.ones((_DIM, _DIM), jnp.float32), preferred_element_type=jnp.float32
    )
    out_ref[...] = (p / denom).reshape(_BB, _LB, _DIM)


@jax.jit
def kernel(x_true, x_pred, masks, ind_of_ind):
    batch, seq, dim = x_true.shape
    grid = (batch // _BB, pl.cdiv(seq, _LB))
    return pl.pallas_call(
        _cl_kernel,
        grid=grid,
        in_specs=[
            pl.BlockSpec((1, _DIM), lambda b, l: (0, 0)),
            pl.BlockSpec((_NUM_MASKS, _DIM), lambda b, l: (0, 0)),
            pl.BlockSpec((_BB, _LB, _DIM), lambda b, l: (b, l, 0)),
            pl.BlockSpec((_BB, _LB, _DIM), lambda b, l: (b, l, 0)),
        ],
        out_specs=pl.BlockSpec((_BB, _LB, _DIM), lambda b, l: (b, l, 0)),
        out_shape=jax.ShapeDtypeStruct(x_true.shape, jnp.float32),
        compiler_params=pltpu.CompilerParams(
            dimension_semantics=("parallel", "parallel"),
        ),
    )(
        ind_of_ind.reshape(1, _DIM).astype(jnp.int32),
        masks,
        x_true,
        x_pred,
    )


# speedup vs baseline: 7.8562x; 1.0173x over previous
import jax
import jax.numpy as jnp
from jax import lax
from jax.experimental import pallas as pl
from jax.experimental.pallas import tpu as pltpu

_MAX_LEN = 199
_DIM = 128
_NUM_MASKS = 32
_BB = 512
_LB = 16


def _cl_kernel(ind_ref, masks_ref, xt_ref, xp_ref, out_ref):
    ind = ind_ref[...]
    kiota = lax.broadcasted_iota(jnp.int32, (_NUM_MASKS, _DIM), 0)
    onehot_kd = (kiota == jnp.broadcast_to(ind, (_NUM_MASKS, _DIM))).astype(
        jnp.float32
    )
    table = lax.dot_general(
        onehot_kd,
        masks_ref[...],
        dimension_numbers=(((0,), (0,)), ((), ())),
        preferred_element_type=jnp.float32,
    )

    rows = _BB * _LB
    xt = xt_ref[...].reshape(rows, _DIM)
    m = jnp.max(xt, axis=-1, keepdims=True)
    eqf = (xt == m).astype(jnp.float32)
    r = lax.broadcasted_iota(jnp.int32, (_DIM, _DIM), 0)
    c = lax.broadcasted_iota(jnp.int32, (_DIM, _DIM), 1)
    tri = (r < c).astype(jnp.float32)
    pf = jnp.dot(eqf, tri, preferred_element_type=jnp.float32)
    onehot = jnp.where(pf == 0.0, eqf, 0.0)

    mask_rows = jnp.dot(
        onehot, table, preferred_element_type=jnp.float32
    )
    p = jnp.exp(xp_ref[...]).reshape(rows, _DIM) * mask_rows
    denom = jnp.dot(
        p, jnp.ones((_DIM, _DIM), jnp.float32), preferred_element_type=jnp.float32
    )
    out_ref[...] = (p / denom).reshape(_BB, _LB, _DIM)


@jax.jit
def kernel(x_true, x_pred, masks, ind_of_ind):
    batch, seq, dim = x_true.shape
    grid = (batch // _BB, pl.cdiv(seq, _LB))
    return pl.pallas_call(
        _cl_kernel,
        grid=grid,
        in_specs=[
            pl.BlockSpec((1, _DIM), lambda b, l: (0, 0)),
            pl.BlockSpec((_NUM_MASKS, _DIM), lambda b, l: (0, 0)),
            pl.BlockSpec((_BB, _LB, _DIM), lambda b, l: (b, l, 0)),
            pl.BlockSpec((_BB, _LB, _DIM), lambda b, l: (b, l, 0)),
        ],
        out_specs=pl.BlockSpec((_BB, _LB, _DIM), lambda b, l: (b, l, 0)),
        out_shape=jax.ShapeDtypeStruct(x_true.shape, jnp.float32),
        compiler_params=pltpu.CompilerParams(
            dimension_semantics=("parallel", "parallel"),
        ),
    )(
        ind_of_ind.reshape(1, _DIM).astype(jnp.int32),
        masks,
        x_true,
        x_pred,
    )
